# Initial kernel scaffold; baseline (speedup 1.0000x reference)
import functools

import jax
import jax.numpy as jnp
from jax import lax
from jax.experimental import pallas as pl
from jax.experimental.pallas import tpu as pltpu
from jax.experimental.pallas import tpu_sc as plsc

N = 10000
D = 128
E = 320000
NC = 2
NS = 16
EP = 327680
ROWS_ALL = EP // 128
ROWS_PER_TILE = ROWS_ALL // (NC * NS)
ROWS_PER_TILE_DEG = ROWS_ALL // NS
NPAD = 10240
SLICE = N // NS


def _fast_rsqrt(d):
    magic = jnp.full((16,), 0x5F3759DF, jnp.int32)
    half = jnp.full((16,), 0.5, jnp.float32)
    three_half = jnp.full((16,), 1.5, jnp.float32)
    i = magic - lax.shift_right_logical(plsc.bitcast(d, jnp.int32), 1)
    y = plsc.bitcast(i, jnp.float32)
    for _ in range(3):
        y = y * (three_half - half * d * y * y)
    return y


_MESH = plsc.VectorSubcoreMesh(
    core_axis_name="c", subcore_axis_name="s", num_cores=NC, num_subcores=NS
)


@functools.partial(
    pl.kernel,
    out_type=(
        jax.ShapeDtypeStruct((2 * N, D), jnp.float32),
        jax.ShapeDtypeStruct((NPAD // 16, 16), jnp.float32),
    ),
    mesh=_MESH,
    scratch_types=(
        pltpu.VMEM((NPAD // 16, 16), jnp.float32),
        pltpu.VMEM((NPAD,), jnp.float32),
        pltpu.VMEM((ROWS_PER_TILE, 128), jnp.int32),
        pltpu.VMEM((ROWS_PER_TILE, 128), jnp.int32),
        pltpu.VMEM((ROWS_PER_TILE, 128), jnp.float32),
        pltpu.VMEM((128, 128), jnp.float32),
        pltpu.VMEM((128,), jnp.float32),
        pltpu.VMEM((NPAD // 16 // 128, 128), jnp.int32),
        pltpu.VMEM_SHARED((N, D), jnp.float32),
        pltpu.VMEM_SHARED((NPAD // 16, 16), jnp.float32),
        pltpu.SemaphoreType.DMA,
    ),
)
def _sc_spmm(x_hbm, src_hbm, dst_hbm, ew_hbm, s_out, deg_out,
             deg_local, dis_local, src_all, dst_all, ew_all, rows, a_buf,
             iidx, S_sh, deg_sh, sem):
    cid = lax.axis_index("c")
    sid = lax.axis_index("s")
    zero16 = jnp.zeros((16,), jnp.float32)
    iota16 = lax.iota(jnp.int32, 16)

    def _zr(r, carry):
        for c8 in range(8):
            rows[r, pl.ds(c8 * 16, 16)] = zero16
        return carry
    lax.fori_loop(0, 128, _zr, 0)

    def _zd(r, carry):
        deg_local[r] = zero16
        return carry
    lax.fori_loop(0, NPAD // 16, _zd, 0)

    def _fi(j, carry):
        iidx[j // 8, pl.ds((j % 8) * 16, 16)] = j * 16 + iota16
        return carry
    lax.fori_loop(0, NPAD // 16 // 16, _fi, 0)

    for k in range(5):
        nrows = 128 if k < 4 else SLICE - 4 * 128
        pltpu.sync_copy(rows.at[pl.ds(0, nrows)],
                        S_sh.at[pl.ds(sid * SLICE + k * 128, nrows)])

    @pl.when(sid == 0)
    def _zero_deg_sh():
        pltpu.sync_copy(deg_local, deg_sh)

    plsc.subcore_barrier()

    def _deg_chunk(it, carry):
        base = sid * ROWS_PER_TILE_DEG + it * ROWS_PER_TILE
        pltpu.sync_copy(dst_hbm.at[pl.ds(base, ROWS_PER_TILE)], src_all)
        pltpu.sync_copy(ew_hbm.at[pl.ds(base, ROWS_PER_TILE)], ew_all)

        def _body(j, c2):
            dv = src_all[j // 8, pl.ds((j % 8) * 16, 16)]
            wv = ew_all[j // 8, pl.ds((j % 8) * 16, 16)]
            plsc.addupdate_scatter(
                deg_local,
                [lax.shift_right_logical(dv, 4), lax.bitwise_and(dv, 15)],
                wv,
            )
            return c2
        lax.fori_loop(0, ROWS_PER_TILE * 8, _body, 0)
        return carry
    lax.fori_loop(0, ROWS_PER_TILE_DEG // ROWS_PER_TILE, _deg_chunk, 0)

    pltpu.sync_copy(deg_local, deg_sh.at[iidx], add=True)
    plsc.subcore_barrier()

    pltpu.sync_copy(deg_sh, deg_local)

    @pl.when(jnp.logical_and(cid == 0, sid == 0))
    def _write_deg():
        pltpu.sync_copy(deg_local, deg_out)

    def _dis(r, carry):
        dis_local[pl.ds(r * 16, 16)] = _fast_rsqrt(deg_local[r])
        return carry
    lax.fori_loop(0, NPAD // 16, _dis, 0)

    tbase = (cid * NS + sid) * ROWS_PER_TILE
    pltpu.sync_copy(src_hbm.at[pl.ds(tbase, ROWS_PER_TILE)], src_all)
    pltpu.sync_copy(dst_hbm.at[pl.ds(tbase, ROWS_PER_TILE)], dst_all)
    pltpu.sync_copy(ew_hbm.at[pl.ds(tbase, ROWS_PER_TILE)], ew_all)

    def _chunk(k, carry):
        pltpu.async_copy(x_hbm.at[src_all.at[k]], rows, sem).wait()

        def _av(v, c2):
            sv = src_all[k, pl.ds(v * 16, 16)]
            wv = ew_all[k, pl.ds(v * 16, 16)]
            a_buf[pl.ds(v * 16, 16)] = plsc.load_gather(dis_local, [sv]) * wv
            return c2
        lax.fori_loop(0, 8, _av, 0)

        def _rowb(r, c2):
            spl = plsc.load_gather(a_buf, [lax.broadcast(r, (16,))])
            for c8 in range(8):
                rows[r, pl.ds(c8 * 16, 16)] = rows[r, pl.ds(c8 * 16, 16)] * spl
            return c2
        lax.fori_loop(0, 128, _rowb, 0)

        pltpu.sync_copy(rows, S_sh.at[dst_all.at[k]], add=True)
        return carry
    lax.fori_loop(0, ROWS_PER_TILE, _chunk, 0)

    plsc.subcore_barrier()

    pltpu.sync_copy(S_sh.at[pl.ds(sid * SLICE, SLICE)],
                    s_out.at[pl.ds(cid * N + sid * SLICE, SLICE)])


BLK = 1000


def _tc_body(s0_ref, s1_ref, x_ref, deg_ref, th_ref, o_ref):
    dis = lax.rsqrt(deg_ref[...])
    x = x_ref[...]
    w = dis * (s0_ref[...] + s1_ref[...] + dis * x)
    o_ref[...] = jnp.dot(w, th_ref[...], preferred_element_type=jnp.float32) * x


_tc_finish = pl.pallas_call(
    _tc_body,
    grid=(N // BLK,),
    in_specs=[
        pl.BlockSpec((BLK, D), lambda i: (i, 0)),
        pl.BlockSpec((BLK, D), lambda i: (i, 0)),
        pl.BlockSpec((BLK, D), lambda i: (i, 0)),
        pl.BlockSpec((BLK, 1), lambda i: (i, 0)),
        pl.BlockSpec((D, D), lambda i: (0, 0)),
    ],
    out_specs=pl.BlockSpec((BLK, D), lambda i: (i, 0)),
    out_shape=jax.ShapeDtypeStruct((N, D), jnp.float32),
)


def kernel(X, edge_index, edge_weight, theta):
    src = edge_index[0].astype(jnp.int32)
    dst = edge_index[1].astype(jnp.int32)
    ew = edge_weight.astype(jnp.float32)
    pad = EP - E
    src2d = jnp.concatenate([src, jnp.zeros((pad,), jnp.int32)]).reshape(ROWS_ALL, 128)
    dst2d = jnp.concatenate([dst, jnp.zeros((pad,), jnp.int32)]).reshape(ROWS_ALL, 128)
    ew2d = jnp.concatenate([ew, jnp.zeros((pad,), jnp.float32)]).reshape(ROWS_ALL, 128)

    s_cat, deg_pad = _sc_spmm(X, src2d, dst2d, ew2d)
    deg = deg_pad.reshape(-1)[:N].reshape(N, 1)
    return _tc_finish(s_cat[:N], s_cat[N:], X, deg, theta)


# speedup vs baseline: 8.7719x; 8.7719x over previous
import functools

import jax
import jax.numpy as jnp
from jax import lax
from jax.experimental import pallas as pl
from jax.experimental.pallas import tpu as pltpu
from jax.experimental.pallas import tpu_sc as plsc

N = 10000
D = 128
E = 320000
NC = 2
NS = 16
EP = 327680
ROWS_ALL = EP // 128
ROWS_PER_TILE = ROWS_ALL // (NC * NS)
ROWS_PER_TILE_DEG = ROWS_ALL // NS
NPAD = 10240
SLICE = 624
DEG_RND = 4
DCH = NPAD // DEG_RND
DSL = DCH // NS


def _fast_rsqrt(d):
    magic = jnp.full((16,), 0x5F3759DF, jnp.int32)
    half = jnp.full((16,), 0.5, jnp.float32)
    three_half = jnp.full((16,), 1.5, jnp.float32)
    i = magic - lax.shift_right_logical(plsc.bitcast(d, jnp.int32), 1)
    y = plsc.bitcast(i, jnp.float32)
    for _ in range(3):
        y = y * (three_half - half * d * y * y)
    return y


_MESH = plsc.VectorSubcoreMesh(
    core_axis_name="c", subcore_axis_name="s", num_cores=NC, num_subcores=NS
)


@functools.partial(
    pl.kernel,
    out_type=(
        jax.ShapeDtypeStruct((2 * N, D), jnp.float32),
        jax.ShapeDtypeStruct((NPAD,), jnp.float32),
    ),
    mesh=_MESH,
    compiler_params=pltpu.CompilerParams(needs_layout_passes=False),
    scratch_types=(
        pltpu.VMEM((NPAD,), jnp.float32),
        pltpu.VMEM((16, 128), jnp.int32),
        pltpu.VMEM((16, 128), jnp.int32),
        pltpu.VMEM((16, 128), jnp.float32),
        pltpu.VMEM((128, 128), jnp.float32),
        pltpu.VMEM((128,), jnp.float32),
        pltpu.VMEM((DSL,), jnp.float32),
        pltpu.VMEM((DSL,), jnp.float32),
        pltpu.VMEM_SHARED((N, D), jnp.float32),
        pltpu.VMEM_SHARED((NS * DCH,), jnp.float32),
        pltpu.VMEM_SHARED((NPAD,), jnp.float32),
        pltpu.SemaphoreType.DMA,
    ),
)
def _sc_spmm(x_hbm, src_hbm, dst_hbm, ew_hbm, s_out, deg_out,
             dd, sidx, didx, ewb, rows, a_buf, tmp_slc, acc_slc,
             S_sh, deg_parts, deg_sh, sem):
    cid = lax.axis_index("c")
    sid = lax.axis_index("s")
    zero16 = jnp.zeros((16,), jnp.float32)

    def _zr(r, carry):
        for c8 in range(8):
            rows[r, pl.ds(c8 * 16, 16)] = zero16
        return carry
    lax.fori_loop(0, 128, _zr, 0)

    def _zd(r, carry):
        dd[pl.ds(r * 16, 16)] = zero16
        return carry
    lax.fori_loop(0, NPAD // 16, _zd, 0)

    for k in range(5):
        nrows = 128 if k < 4 else SLICE - 4 * 128
        pltpu.sync_copy(rows.at[pl.ds(0, nrows)],
                        S_sh.at[pl.ds(sid * SLICE + k * 128, nrows)])

    @pl.when(sid == NS - 1)
    def _zero_tail():
        pltpu.sync_copy(rows.at[pl.ds(0, 16)], S_sh.at[pl.ds(NS * SLICE, 16)])

    def _deg_slab(it, carry):
        base = sid * ROWS_PER_TILE_DEG + it * 16
        pltpu.sync_copy(dst_hbm.at[pl.ds(base, 16)], didx)
        pltpu.sync_copy(ew_hbm.at[pl.ds(base, 16)], ewb)

        def _body(j, c2):
            dv = didx[j // 8, pl.ds((j % 8) * 16, 16)]
            wv = ewb[j // 8, pl.ds((j % 8) * 16, 16)]
            plsc.addupdate_scatter(dd, [dv], wv)
            return c2
        lax.fori_loop(0, 128, _body, 0)
        return carry
    lax.fori_loop(0, ROWS_PER_TILE_DEG // 16, _deg_slab, 0)

    for rch in range(DEG_RND):
        pltpu.sync_copy(dd.at[pl.ds(rch * DCH, DCH)],
                        deg_parts.at[pl.ds(sid * DCH, DCH)])
        plsc.subcore_barrier()

        def _za(r, carry):
            acc_slc[pl.ds(r * 16, 16)] = zero16
            return carry
        lax.fori_loop(0, DSL // 16, _za, 0)

        def _accp(p, carry):
            pltpu.sync_copy(deg_parts.at[pl.ds(p * DCH + sid * DSL, DSL)],
                            tmp_slc)

            def _r(r, c2):
                acc_slc[pl.ds(r * 16, 16)] = (
                    acc_slc[pl.ds(r * 16, 16)] + tmp_slc[pl.ds(r * 16, 16)])
                return c2
            lax.fori_loop(0, DSL // 16, _r, 0)
            return carry
        lax.fori_loop(0, NS, _accp, 0)

        pltpu.sync_copy(acc_slc, deg_sh.at[pl.ds(rch * DCH + sid * DSL, DSL)])
        plsc.subcore_barrier()

    pltpu.sync_copy(deg_sh, dd)

    one16 = jnp.ones((16,), jnp.float32)

    def _p1(r, carry):
        dd[pl.ds(r * 16, 16)] = dd[pl.ds(r * 16, 16)] + one16
        return carry
    lax.fori_loop(0, NPAD // 16, _p1, 0)

    @pl.when(jnp.logical_and(cid == 0, sid == 0))
    def _write_deg():
        pltpu.sync_copy(dd, deg_out)

    def _dis(r, carry):
        dd[pl.ds(r * 16, 16)] = _fast_rsqrt(dd[pl.ds(r * 16, 16)])
        return carry
    lax.fori_loop(0, NPAD // 16, _dis, 0)

    tbase = (cid * NS + sid) * ROWS_PER_TILE

    def _grp(g, carry):
        gb = tbase + g * 16
        pltpu.sync_copy(src_hbm.at[pl.ds(gb, 16)], sidx)
        pltpu.sync_copy(dst_hbm.at[pl.ds(gb, 16)], didx)
        pltpu.sync_copy(ew_hbm.at[pl.ds(gb, 16)], ewb)

        def _chunk(k, c1):
            pltpu.async_copy(x_hbm.at[sidx.at[k]], rows, sem).wait()

            def _av(v, c2):
                sv = sidx[k, pl.ds(v * 16, 16)]
                wv = ewb[k, pl.ds(v * 16, 16)]
                a_buf[pl.ds(v * 16, 16)] = plsc.load_gather(dd, [sv]) * wv
                return c2
            lax.fori_loop(0, 8, _av, 0)

            def _rowb(r, c2):
                spl = plsc.load_gather(a_buf, [lax.broadcast(r, (16,))])
                for c8 in range(8):
                    rows[r, pl.ds(c8 * 16, 16)] = (
                        rows[r, pl.ds(c8 * 16, 16)] * spl)
                return c2
            lax.fori_loop(0, 128, _rowb, 0)

            pltpu.sync_copy(rows, S_sh.at[didx.at[k]], add=True)
            return c1
        lax.fori_loop(0, 16, _chunk, 0)
        return carry
    lax.fori_loop(0, ROWS_PER_TILE // 16, _grp, 0)

    plsc.subcore_barrier()

    for k in range(5):
        nrows = 128 if k < 4 else SLICE - 4 * 128
        off = sid * SLICE + k * 128
        pltpu.sync_copy(S_sh.at[pl.ds(off, nrows)],
                        s_out.at[pl.ds(cid * N + off, nrows)])

    @pl.when(sid == NS - 1)
    def _export_tail():
        pltpu.sync_copy(S_sh.at[pl.ds(NS * SLICE, 16)],
                        s_out.at[pl.ds(cid * N + NS * SLICE, 16)])


BLK = 1000


def _tc_body(s0_ref, s1_ref, x_ref, deg_ref, th_ref, o_ref):
    dis = lax.rsqrt(deg_ref[...])
    x = x_ref[...]
    w = dis * (s0_ref[...] + s1_ref[...] + dis * x)
    o_ref[...] = jnp.dot(w, th_ref[...], preferred_element_type=jnp.float32) * x


_tc_finish = pl.pallas_call(
    _tc_body,
    grid=(N // BLK,),
    in_specs=[
        pl.BlockSpec((BLK, D), lambda i: (i, 0)),
        pl.BlockSpec((BLK, D), lambda i: (i, 0)),
        pl.BlockSpec((BLK, D), lambda i: (i, 0)),
        pl.BlockSpec((BLK, 1), lambda i: (i, 0)),
        pl.BlockSpec((D, D), lambda i: (0, 0)),
    ],
    out_specs=pl.BlockSpec((BLK, D), lambda i: (i, 0)),
    out_shape=jax.ShapeDtypeStruct((N, D), jnp.float32),
)


def kernel(X, edge_index, edge_weight, theta):
    src = edge_index[0].astype(jnp.int32)
    dst = edge_index[1].astype(jnp.int32)
    ew = edge_weight.astype(jnp.float32)
    pad = EP - E
    src2d = jnp.concatenate([src, jnp.zeros((pad,), jnp.int32)]).reshape(ROWS_ALL, 128)
    dst2d = jnp.concatenate([dst, jnp.zeros((pad,), jnp.int32)]).reshape(ROWS_ALL, 128)
    ew2d = jnp.concatenate([ew, jnp.zeros((pad,), jnp.float32)]).reshape(ROWS_ALL, 128)

    s_cat, deg_pad = _sc_spmm(X, src2d, dst2d, ew2d)
    deg = deg_pad[:N].reshape(N, 1)
    return _tc_finish(s_cat[:N], s_cat[N:], X, deg, theta)


# speedup vs baseline: 8.9059x; 1.0153x over previous
import functools

import jax
import jax.numpy as jnp
from jax import lax
from jax.experimental import pallas as pl
from jax.experimental.pallas import tpu as pltpu
from jax.experimental.pallas import tpu_sc as plsc

N = 10000
D = 128
E = 320000
NC = 2
NS = 16
EP = 327680
ROWS_ALL = EP // 128
ROWS_PER_TILE = ROWS_ALL // (NC * NS)
R0 = 160
R1 = (ROWS_ALL - NS * R0) // NS
ROWS_PER_TILE_DEG = ROWS_ALL // NS
NPAD = 10240
SLICE = 624
DEG_RND = 10
DCH = NPAD // DEG_RND
DSL = DCH // NS


def _fast_rsqrt(d):
    magic = jnp.full((16,), 0x5F3759DF, jnp.int32)
    half = jnp.full((16,), 0.5, jnp.float32)
    three_half = jnp.full((16,), 1.5, jnp.float32)
    i = magic - lax.shift_right_logical(plsc.bitcast(d, jnp.int32), 1)
    y = plsc.bitcast(i, jnp.float32)
    for _ in range(3):
        y = y * (three_half - half * d * y * y)
    return y


_MESH = plsc.VectorSubcoreMesh(
    core_axis_name="c", subcore_axis_name="s", num_cores=NC, num_subcores=NS
)


@functools.partial(
    pl.kernel,
    out_type=(
        jax.ShapeDtypeStruct((N, D), jnp.float32),
        jax.ShapeDtypeStruct((NPAD,), jnp.float32),
    ),
    mesh=_MESH,
    compiler_params=pltpu.CompilerParams(needs_layout_passes=False),
    scratch_types=(
        pltpu.VMEM((NPAD,), jnp.float32),
        pltpu.VMEM((16, 128), jnp.int32),
        pltpu.VMEM((16, 128), jnp.int32),
        pltpu.VMEM((16, 128), jnp.float32),
        pltpu.VMEM((128, 128), jnp.float32),
        pltpu.VMEM((128, 128), jnp.float32),
        pltpu.VMEM((DSL,), jnp.float32),
        pltpu.VMEM((DSL,), jnp.float32),
        pltpu.VMEM_SHARED((N, D), jnp.float32),
        pltpu.VMEM_SHARED((NS * DCH,), jnp.float32),
        pltpu.VMEM_SHARED((NPAD,), jnp.float32),
        pltpu.SemaphoreType.DMA,
        pltpu.SemaphoreType.DMA,
        pltpu.SemaphoreType.DMA,
        pltpu.SemaphoreType.DMA,
        pltpu.SemaphoreType.DMA,
    ),
)
def _sc_spmm(x_hbm, src_hbm, dst_hbm, dstf_hbm, ew_hbm, s_out, deg_out,
             dd, sidx, didx, ewb, rows, rows_b, tmp_slc, acc_slc,
             S_sh, deg_parts, deg_sh, gsem_a, gsem_b, ssem_a, ssem_b, isem):
    cid = lax.axis_index("c")
    sid = lax.axis_index("s")
    zero16 = jnp.zeros((16,), jnp.float32)

    def _zr(r, carry):
        for c8 in range(8):
            rows[r, pl.ds(c8 * 16, 16)] = zero16
        return carry
    lax.fori_loop(0, 128, _zr, 0)

    def _zd(r, carry):
        dd[pl.ds(r * 16, 16)] = zero16
        return carry
    lax.fori_loop(0, NPAD // 16, _zd, 0)

    for k in range(5):
        nrows = 128 if k < 4 else SLICE - 4 * 128
        pltpu.sync_copy(rows.at[pl.ds(0, nrows)],
                        S_sh.at[pl.ds(sid * SLICE + k * 128, nrows)])

    @pl.when(sid == NS - 1)
    def _zero_tail():
        pltpu.sync_copy(rows.at[pl.ds(0, 16)], S_sh.at[pl.ds(NS * SLICE, 16)])

    for half in range(2):
        nrows = 128 if half == 0 else ROWS_PER_TILE_DEG - 128
        base = sid * ROWS_PER_TILE_DEG + half * 128
        d1 = pltpu.async_copy(dstf_hbm.at[pl.ds(base, nrows)],
                              rows.at[pl.ds(0, nrows)], gsem_a)
        d2 = pltpu.async_copy(ew_hbm.at[pl.ds(base, nrows)],
                              rows_b.at[pl.ds(0, nrows)], gsem_b)
        d1.wait()
        d2.wait()

        def _body(j, c2):
            dv = plsc.bitcast(rows[j // 8, pl.ds((j % 8) * 16, 16)], jnp.int32)
            wv = rows_b[j // 8, pl.ds((j % 8) * 16, 16)]
            plsc.addupdate_scatter(dd, [dv], wv)
            return c2
        lax.fori_loop(0, nrows * 8, _body, 0)

    for rch in range(DEG_RND):
        pltpu.sync_copy(dd.at[pl.ds(rch * DCH, DCH)],
                        deg_parts.at[pl.ds(sid * DCH, DCH)])
        plsc.subcore_barrier()

        def _za(r, carry):
            acc_slc[pl.ds(r * 16, 16)] = zero16
            return carry
        lax.fori_loop(0, DSL // 16, _za, 0)

        def _accp(p, carry):
            pltpu.sync_copy(deg_parts.at[pl.ds(p * DCH + sid * DSL, DSL)],
                            tmp_slc)

            def _r(r, c2):
                acc_slc[pl.ds(r * 16, 16)] = (
                    acc_slc[pl.ds(r * 16, 16)] + tmp_slc[pl.ds(r * 16, 16)])
                return c2
            lax.fori_loop(0, DSL // 16, _r, 0)
            return carry
        lax.fori_loop(0, NS, _accp, 0)

        pltpu.sync_copy(acc_slc, deg_sh.at[pl.ds(rch * DCH + sid * DSL, DSL)])
        plsc.subcore_barrier()

    pltpu.sync_copy(deg_sh, dd)

    one16 = jnp.ones((16,), jnp.float32)

    def _p1(r, carry):
        dd[pl.ds(r * 16, 16)] = dd[pl.ds(r * 16, 16)] + one16
        return carry
    lax.fori_loop(0, NPAD // 16, _p1, 0)

    @pl.when(jnp.logical_and(cid == 0, sid == 0))
    def _write_deg():
        pltpu.sync_copy(dd, deg_out)

    def _dis(r, carry):
        dd[pl.ds(r * 16, 16)] = _fast_rsqrt(dd[pl.ds(r * 16, 16)])
        return carry
    lax.fori_loop(0, NPAD // 16, _dis, 0)

    tbase = jnp.where(cid == 0, sid * R0, NS * R0 + sid * R1)
    ngrp = jnp.where(cid == 0, R0 // 16, R1 // 16)
    bufs = (rows, rows_b)
    gsems = (gsem_a, gsem_b)
    ssems = (ssem_a, ssem_b)

    def _scale(buf, b):
        def _rowg(g16, c2):
            sv = sidx[b, pl.ds(g16 * 16, 16)]
            wv = ewb[b, pl.ds(g16 * 16, 16)]
            av = plsc.load_gather(dd, [sv]) * wv
            for u in range(16):
                spl = lax.broadcast(av[u], (16,))
                r = g16 * 16 + u
                for c8 in range(8):
                    buf[r, pl.ds(c8 * 16, 16)] = (
                        buf[r, pl.ds(c8 * 16, 16)] * spl)
            return c2
        lax.fori_loop(0, 8, _rowg, 0)

    NB = 16

    def _grp(g, carry):
        gb = tbase + g * NB
        i1 = pltpu.async_copy(src_hbm.at[pl.ds(gb, NB)], sidx, isem)
        i2 = pltpu.async_copy(dst_hbm.at[pl.ds(gb, NB)], didx, gsems[1])
        i3 = pltpu.async_copy(ew_hbm.at[pl.ds(gb, NB)], ewb, ssems[1])
        i1.wait()
        i2.wait()
        i3.wait()

        gather_d = pltpu.async_copy(x_hbm.at[sidx.at[0]], bufs[0], gsems[0])
        scat_d = [None, None]
        for b in range(NB):
            i, o = b % 2, (b + 1) % 2
            if b < NB - 1:
                if scat_d[o] is not None:
                    scat_d[o].wait()
                next_gather = pltpu.async_copy(
                    x_hbm.at[sidx.at[b + 1]], bufs[o], gsems[o])
            gather_d.wait()
            _scale(bufs[i], b)
            scat_d[i] = pltpu.async_copy(
                bufs[i], S_sh.at[didx.at[b]], ssems[i], add=True)
            if b < NB - 1:
                gather_d = next_gather
        scat_d[0].wait()
        scat_d[1].wait()
        return carry
    lax.fori_loop(0, ngrp, _grp, 0)

    plsc.subcore_barrier()

    @pl.when(cid == 0)
    def _export():
        exp = []
        for k in range(5):
            nrows = 128 if k < 4 else SLICE - 4 * 128
            off = sid * SLICE + k * 128
            exp.append(pltpu.async_copy(S_sh.at[pl.ds(off, nrows)],
                                        s_out.at[pl.ds(off, nrows)],
                                        gsem_a))

        @pl.when(sid == NS - 1)
        def _export_tail():
            pltpu.sync_copy(S_sh.at[pl.ds(NS * SLICE, 16)],
                            s_out.at[pl.ds(NS * SLICE, 16)])

        for d in exp:
            d.wait()


BLK = 1000


def _tc_body(s0_ref, x_ref, deg_ref, th_ref, o_ref):
    dis = lax.rsqrt(deg_ref[...])
    x = x_ref[...]
    w = dis * (s0_ref[...] + dis * x)
    o_ref[...] = jnp.dot(w, th_ref[...], preferred_element_type=jnp.float32) * x


_tc_finish = pl.pallas_call(
    _tc_body,
    grid=(N // BLK,),
    in_specs=[
        pl.BlockSpec((BLK, D), lambda i: (i, 0)),
        pl.BlockSpec((BLK, D), lambda i: (i, 0)),
        pl.BlockSpec((BLK, 1), lambda i: (i, 0)),
        pl.BlockSpec((D, D), lambda i: (0, 0)),
    ],
    out_specs=pl.BlockSpec((BLK, D), lambda i: (i, 0)),
    out_shape=jax.ShapeDtypeStruct((N, D), jnp.float32),
)


def kernel(X, edge_index, edge_weight, theta):
    src = edge_index[0].astype(jnp.int32)
    dst = edge_index[1].astype(jnp.int32)
    ew = edge_weight.astype(jnp.float32)
    pad = EP - E
    src2d = jnp.concatenate([src, jnp.zeros((pad,), jnp.int32)]).reshape(ROWS_ALL, 128)
    dst2d = jnp.concatenate([dst, jnp.zeros((pad,), jnp.int32)]).reshape(ROWS_ALL, 128)
    ew2d = jnp.concatenate([ew, jnp.zeros((pad,), jnp.float32)]).reshape(ROWS_ALL, 128)

    dstf2d = jax.lax.bitcast_convert_type(dst2d, jnp.float32)
    s0, deg_pad = _sc_spmm(X, src2d, dst2d, dstf2d, ew2d)
    deg = deg_pad[:N].reshape(N, 1)
    return _tc_finish(s0, X, deg, theta)


# speedup vs baseline: 11.4703x; 1.2879x over previous
import functools

import jax
import jax.numpy as jnp
from jax import lax
from jax.experimental import pallas as pl
from jax.experimental.pallas import tpu as pltpu
from jax.experimental.pallas import tpu_sc as plsc

N = 10000
D = 128
E = 320000
NC = 2
NS = 16
EP = 327680
ROWS_ALL = EP // 128
ROWS_PER_TILE = ROWS_ALL // (NC * NS)
R0 = 128
R1 = (ROWS_ALL - NS * R0) // NS
ROWS_PER_TILE_DEG = ROWS_ALL // NS
NPAD = 10240
SLICE = 624
DEG_RND = 10
DCH = NPAD // DEG_RND
DSL = DCH // NS


def _fast_rsqrt(d):
    magic = jnp.full((16,), 0x5F3759DF, jnp.int32)
    half = jnp.full((16,), 0.5, jnp.float32)
    three_half = jnp.full((16,), 1.5, jnp.float32)
    i = magic - lax.shift_right_logical(plsc.bitcast(d, jnp.int32), 1)
    y = plsc.bitcast(i, jnp.float32)
    for _ in range(3):
        y = y * (three_half - half * d * y * y)
    return y


_MESH = plsc.VectorSubcoreMesh(
    core_axis_name="c", subcore_axis_name="s", num_cores=NC, num_subcores=NS
)


@functools.partial(
    pl.kernel,
    out_type=(
        jax.ShapeDtypeStruct((N, D), jnp.float32),
        jax.ShapeDtypeStruct((N // 2, D), jnp.float32),
        jax.ShapeDtypeStruct((NPAD,), jnp.float32),
    ),
    mesh=_MESH,
    compiler_params=pltpu.CompilerParams(needs_layout_passes=False),
    scratch_types=(
        pltpu.VMEM((NPAD,), jnp.float32),
        pltpu.VMEM((16, 128), jnp.int32),
        pltpu.VMEM((16, 128), jnp.int32),
        pltpu.VMEM((16, 128), jnp.float32),
        pltpu.VMEM((128, 128), jnp.float32),
        pltpu.VMEM((128, 128), jnp.float32),
        pltpu.VMEM((DSL,), jnp.float32),
        pltpu.VMEM((DSL,), jnp.float32),
        pltpu.VMEM_SHARED((N, D), jnp.float32),
        pltpu.VMEM_SHARED((NS * DCH,), jnp.float32),
        pltpu.VMEM_SHARED((NPAD,), jnp.float32),
        pltpu.SemaphoreType.DMA,
        pltpu.SemaphoreType.DMA,
        pltpu.SemaphoreType.DMA,
        pltpu.SemaphoreType.DMA,
        pltpu.SemaphoreType.DMA,
    ),
)
def _sc_spmm(x_hbm, src_hbm, dst_hbm, dstf_hbm, ew_hbm, s_out, s1_out, deg_out,
             dd, sidx, didx, ewb, rows, rows_b, tmp_slc, acc_slc,
             S_sh, deg_parts, deg_sh, gsem_a, gsem_b, ssem_a, ssem_b, isem):
    cid = lax.axis_index("c")
    sid = lax.axis_index("s")
    zero16 = jnp.zeros((16,), jnp.float32)

    def _zr(r, carry):
        for c8 in range(8):
            rows[r, pl.ds(c8 * 16, 16)] = zero16
        return carry
    lax.fori_loop(0, 128, _zr, 0)

    def _zd(r, carry):
        dd[pl.ds(r * 16, 16)] = zero16
        return carry
    lax.fori_loop(0, NPAD // 16, _zd, 0)

    for k in range(5):
        nrows = 128 if k < 4 else SLICE - 4 * 128
        pltpu.sync_copy(rows.at[pl.ds(0, nrows)],
                        S_sh.at[pl.ds(sid * SLICE + k * 128, nrows)])

    @pl.when(sid == NS - 1)
    def _zero_tail():
        pltpu.sync_copy(rows.at[pl.ds(0, 16)], S_sh.at[pl.ds(NS * SLICE, 16)])

    for half in range(2):
        nrows = 128 if half == 0 else ROWS_PER_TILE_DEG - 128
        base = sid * ROWS_PER_TILE_DEG + half * 128
        d1 = pltpu.async_copy(dstf_hbm.at[pl.ds(base, nrows)],
                              rows.at[pl.ds(0, nrows)], gsem_a)
        d2 = pltpu.async_copy(ew_hbm.at[pl.ds(base, nrows)],
                              rows_b.at[pl.ds(0, nrows)], gsem_b)
        d1.wait()
        d2.wait()

        def _body(j, c2):
            dv = plsc.bitcast(rows[j // 8, pl.ds((j % 8) * 16, 16)], jnp.int32)
            wv = rows_b[j // 8, pl.ds((j % 8) * 16, 16)]
            plsc.addupdate_scatter(dd, [dv], wv)
            return c2
        lax.fori_loop(0, nrows * 8, _body, 0)

    for rch in range(DEG_RND):
        pltpu.sync_copy(dd.at[pl.ds(rch * DCH, DCH)],
                        deg_parts.at[pl.ds(sid * DCH, DCH)])
        plsc.subcore_barrier()

        def _za(r, carry):
            acc_slc[pl.ds(r * 16, 16)] = zero16
            return carry
        lax.fori_loop(0, DSL // 16, _za, 0)

        def _accp(p, carry):
            pltpu.sync_copy(deg_parts.at[pl.ds(p * DCH + sid * DSL, DSL)],
                            tmp_slc)

            def _r(r, c2):
                acc_slc[pl.ds(r * 16, 16)] = (
                    acc_slc[pl.ds(r * 16, 16)] + tmp_slc[pl.ds(r * 16, 16)])
                return c2
            lax.fori_loop(0, DSL // 16, _r, 0)
            return carry
        lax.fori_loop(0, NS, _accp, 0)

        pltpu.sync_copy(acc_slc, deg_sh.at[pl.ds(rch * DCH + sid * DSL, DSL)])
        plsc.subcore_barrier()

    pltpu.sync_copy(deg_sh, dd)

    one16 = jnp.ones((16,), jnp.float32)

    def _p1(r, carry):
        dd[pl.ds(r * 16, 16)] = dd[pl.ds(r * 16, 16)] + one16
        return carry
    lax.fori_loop(0, NPAD // 16, _p1, 0)

    @pl.when(jnp.logical_and(cid == 0, sid == 0))
    def _write_deg():
        pltpu.sync_copy(dd, deg_out)

    def _dis(r, carry):
        dd[pl.ds(r * 16, 16)] = _fast_rsqrt(dd[pl.ds(r * 16, 16)])
        return carry
    lax.fori_loop(0, NPAD // 16, _dis, 0)

    tbase = jnp.where(cid == 0, sid * R0, NS * R0 + sid * R1)
    ngrp = jnp.where(cid == 0, R0 // 16, R1 // 16)
    bufs = (rows, rows_b)
    gsems = (gsem_a, gsem_b)
    ssems = (ssem_a, ssem_b)

    def _scale(buf, b):
        def _rowg(g16, c2):
            sv = sidx[b, pl.ds(g16 * 16, 16)]
            wv = ewb[b, pl.ds(g16 * 16, 16)]
            av = plsc.load_gather(dd, [sv]) * wv
            for u in range(16):
                spl = lax.broadcast(av[u], (16,))
                r = g16 * 16 + u
                for c8 in range(8):
                    buf[r, pl.ds(c8 * 16, 16)] = (
                        buf[r, pl.ds(c8 * 16, 16)] * spl)
            return c2
        lax.fori_loop(0, 8, _rowg, 0)

    NB = 16

    def _grp(g, carry):
        gb = tbase + g * NB
        i1 = pltpu.async_copy(src_hbm.at[pl.ds(gb, NB)], sidx, isem)
        i2 = pltpu.async_copy(dst_hbm.at[pl.ds(gb, NB)], didx, gsems[1])
        i3 = pltpu.async_copy(ew_hbm.at[pl.ds(gb, NB)], ewb, ssems[1])
        i1.wait()
        i2.wait()
        i3.wait()

        gather_d = pltpu.async_copy(x_hbm.at[sidx.at[0]], bufs[0], gsems[0])
        scat_d = [None, None]
        for b in range(NB):
            i, o = b % 2, (b + 1) % 2
            if b < NB - 1:
                if scat_d[o] is not None:
                    scat_d[o].wait()
                next_gather = pltpu.async_copy(
                    x_hbm.at[sidx.at[b + 1]], bufs[o], gsems[o])
            gather_d.wait()
            _scale(bufs[i], b)
            scat_d[i] = pltpu.async_copy(
                bufs[i], S_sh.at[didx.at[b]], ssems[i], add=True)
            if b < NB - 1:
                gather_d = next_gather
        scat_d[0].wait()
        scat_d[1].wait()
        return carry
    lax.fori_loop(0, ngrp, _grp, 0)

    plsc.subcore_barrier()

    @pl.when(cid == 0)
    def _export0():
        exp = []
        for k in range(5):
            nrows = 128 if k < 4 else SLICE - 4 * 128
            off = sid * SLICE + k * 128
            exp.append(pltpu.async_copy(S_sh.at[pl.ds(off, nrows)],
                                        s_out.at[pl.ds(off, nrows)],
                                        gsem_a))

        @pl.when(sid == NS - 1)
        def _export_tail0():
            pltpu.sync_copy(S_sh.at[pl.ds(NS * SLICE, 16)],
                            s_out.at[pl.ds(NS * SLICE, 16)])

        for d in exp:
            d.wait()

    @pl.when(cid == 1)
    def _export1():
        rnd = jnp.full((16,), 0x8000, jnp.int32)

        def _pack_block(nrows, half):
            def _pr(r2, c2):
                for rr in range(2):
                    for k4 in range(4):
                        va = plsc.bitcast(
                            rows[2 * r2 + rr, pl.ds(k4 * 16, 16)], jnp.int32)
                        vb = plsc.bitcast(
                            rows[2 * r2 + rr, pl.ds(64 + k4 * 16, 16)],
                            jnp.int32)
                        pk = lax.bitwise_or(
                            lax.shift_right_logical(va + rnd, 16),
                            lax.shift_left(
                                lax.shift_right_logical(vb + rnd, 16), 16))
                        rows_b[half * 64 + r2, pl.ds(rr * 64 + k4 * 16, 16)]                             = plsc.bitcast(pk, jnp.float32)
                return c2
            lax.fori_loop(0, nrows // 2, _pr, 0)

        prev = None
        for k in range(5):
            nrows = 128 if k < 4 else SLICE - 4 * 128
            off = sid * SLICE + k * 128
            half = k % 2
            pltpu.sync_copy(S_sh.at[pl.ds(off, nrows)],
                            rows.at[pl.ds(0, nrows)])
            _pack_block(nrows, half)
            if prev is not None:
                prev.wait()
            off2 = sid * (SLICE // 2) + k * 64
            prev = pltpu.async_copy(
                rows_b.at[pl.ds(half * 64, nrows // 2)],
                s1_out.at[pl.ds(off2, nrows // 2)], ssems[half])
        prev.wait()

        @pl.when(sid == NS - 1)
        def _export_tail1():
            pltpu.sync_copy(S_sh.at[pl.ds(NS * SLICE, 16)],
                            rows.at[pl.ds(0, 16)])
            _pack_block(16, 0)
            pltpu.sync_copy(rows_b.at[pl.ds(0, 8)],
                            s1_out.at[pl.ds(NS * SLICE // 2, 8)])


BLK = 1000


def _tc_body(s0_ref, s1_ref, x_ref, deg_ref, th_ref, o_ref):
    dis = lax.rsqrt(deg_ref[...])
    x = x_ref[...]
    w = dis * (s0_ref[...] + s1_ref[...] + dis * x)
    o_ref[...] = jnp.dot(w, th_ref[...], preferred_element_type=jnp.float32) * x


_tc_finish = pl.pallas_call(
    _tc_body,
    grid=(N // BLK,),
    in_specs=[
        pl.BlockSpec((BLK, D), lambda i: (i, 0)),
        pl.BlockSpec((BLK, D), lambda i: (i, 0)),
        pl.BlockSpec((BLK, D), lambda i: (i, 0)),
        pl.BlockSpec((BLK, 1), lambda i: (i, 0)),
        pl.BlockSpec((D, D), lambda i: (0, 0)),
    ],
    out_specs=pl.BlockSpec((BLK, D), lambda i: (i, 0)),
    out_shape=jax.ShapeDtypeStruct((N, D), jnp.float32),
)


def kernel(X, edge_index, edge_weight, theta):
    src = edge_index[0].astype(jnp.int32)
    dst = edge_index[1].astype(jnp.int32)
    ew = edge_weight.astype(jnp.float32)
    pad = EP - E
    src2d = jnp.concatenate([src, jnp.zeros((pad,), jnp.int32)]).reshape(ROWS_ALL, 128)
    dst2d = jnp.concatenate([dst, jnp.zeros((pad,), jnp.int32)]).reshape(ROWS_ALL, 128)
    ew2d = jnp.concatenate([ew, jnp.zeros((pad,), jnp.float32)]).reshape(ROWS_ALL, 128)

    dstf2d = jax.lax.bitcast_convert_type(dst2d, jnp.float32)
    s0, s1p, deg_pad = _sc_spmm(X, src2d, dst2d, dstf2d, ew2d)
    deg = deg_pad[:N].reshape(N, 1)
    u = jax.lax.bitcast_convert_type(s1p, jnp.uint32).reshape(N, 64)
    lo = jax.lax.bitcast_convert_type(u << 16, jnp.float32)
    hi = jax.lax.bitcast_convert_type((u >> 16) << 16, jnp.float32)
    s1 = jnp.concatenate([lo, hi], axis=1)
    return _tc_finish(s0, s1, X, deg, theta)


# speedup vs baseline: 14.7272x; 1.2839x over previous
import functools

import jax
import jax.numpy as jnp
from jax import lax
from jax.experimental import pallas as pl
from jax.experimental.pallas import tpu as pltpu
from jax.experimental.pallas import tpu_sc as plsc

N = 10000
D = 128
E = 320000
NC = 2
NS = 16
EP = 327680
ROWS_ALL = EP // 128
ROWS_PER_TILE = ROWS_ALL // (NC * NS)
R0 = 144
R1 = (ROWS_ALL - NS * R0) // NS
ROWS_PER_TILE_DEG = ROWS_ALL // NS
NPAD = 10240
SLICE = 624
DEG_RND = 10
DCH = NPAD // DEG_RND
DSL = DCH // NS


def _fast_rsqrt(d):
    magic = jnp.full((16,), 0x5F3759DF, jnp.int32)
    half = jnp.full((16,), 0.5, jnp.float32)
    three_half = jnp.full((16,), 1.5, jnp.float32)
    i = magic - lax.shift_right_logical(plsc.bitcast(d, jnp.int32), 1)
    y = plsc.bitcast(i, jnp.float32)
    for _ in range(3):
        y = y * (three_half - half * d * y * y)
    return y


_MESH = plsc.VectorSubcoreMesh(
    core_axis_name="c", subcore_axis_name="s", num_cores=NC, num_subcores=NS
)


@functools.partial(
    pl.kernel,
    out_type=(
        jax.ShapeDtypeStruct((2 * N, D), jnp.float32),
        jax.ShapeDtypeStruct((NPAD,), jnp.float32),
    ),
    mesh=_MESH,
    compiler_params=pltpu.CompilerParams(needs_layout_passes=False),
    scratch_types=(
        pltpu.VMEM((NPAD,), jnp.float32),
        pltpu.VMEM((16, 128), jnp.int32),
        pltpu.VMEM((16, 128), jnp.int32),
        pltpu.VMEM((16, 128), jnp.float32),
        pltpu.VMEM((128, 128), jnp.float32),
        pltpu.VMEM((128, 128), jnp.float32),
        pltpu.VMEM((DSL,), jnp.float32),
        pltpu.VMEM((DSL,), jnp.float32),
        pltpu.VMEM_SHARED((N, D), jnp.float32),
        pltpu.VMEM_SHARED((NS * DCH,), jnp.float32),
        pltpu.VMEM_SHARED((NPAD,), jnp.float32),
        pltpu.SemaphoreType.DMA,
        pltpu.SemaphoreType.DMA,
        pltpu.SemaphoreType.DMA,
        pltpu.SemaphoreType.DMA,
        pltpu.SemaphoreType.DMA,
    ),
)
def _sc_spmm(x_hbm, src_hbm, dst_hbm, dstf_hbm, ew_hbm, s_out, deg_out,
             dd, sidx, didx, ewb, rows, rows_b, tmp_slc, acc_slc,
             S_sh, deg_parts, deg_sh, gsem_a, gsem_b, ssem_a, ssem_b, isem):
    cid = lax.axis_index("c")
    sid = lax.axis_index("s")
    zero16 = jnp.zeros((16,), jnp.float32)

    def _zr(r, carry):
        for c8 in range(8):
            rows[r, pl.ds(c8 * 16, 16)] = zero16
        return carry
    lax.fori_loop(0, 128, _zr, 0)

    def _zd(r, carry):
        dd[pl.ds(r * 16, 16)] = zero16
        return carry
    lax.fori_loop(0, NPAD // 16, _zd, 0)

    for k in range(5):
        nrows = 128 if k < 4 else SLICE - 4 * 128
        pltpu.sync_copy(rows.at[pl.ds(0, nrows)],
                        S_sh.at[pl.ds(sid * SLICE + k * 128, nrows)])

    @pl.when(sid == NS - 1)
    def _zero_tail():
        pltpu.sync_copy(rows.at[pl.ds(0, 16)], S_sh.at[pl.ds(NS * SLICE, 16)])

    for half in range(2):
        nrows = 128 if half == 0 else ROWS_PER_TILE_DEG - 128
        base = sid * ROWS_PER_TILE_DEG + half * 128
        d1 = pltpu.async_copy(dstf_hbm.at[pl.ds(base, nrows)],
                              rows.at[pl.ds(0, nrows)], gsem_a)
        d2 = pltpu.async_copy(ew_hbm.at[pl.ds(base, nrows)],
                              rows_b.at[pl.ds(0, nrows)], gsem_b)
        d1.wait()
        d2.wait()

        def _body(j, c2):
            dv = plsc.bitcast(rows[j // 8, pl.ds((j % 8) * 16, 16)], jnp.int32)
            wv = rows_b[j // 8, pl.ds((j % 8) * 16, 16)]
            plsc.addupdate_scatter(dd, [dv], wv)
            return c2
        lax.fori_loop(0, nrows * 8, _body, 0)

    for rch in range(DEG_RND):
        pltpu.sync_copy(dd.at[pl.ds(rch * DCH, DCH)],
                        deg_parts.at[pl.ds(sid * DCH, DCH)])
        plsc.subcore_barrier()

        def _za(r, carry):
            acc_slc[pl.ds(r * 16, 16)] = zero16
            return carry
        lax.fori_loop(0, DSL // 16, _za, 0)

        def _accp(p, carry):
            pltpu.sync_copy(deg_parts.at[pl.ds(p * DCH + sid * DSL, DSL)],
                            tmp_slc)

            def _r(r, c2):
                acc_slc[pl.ds(r * 16, 16)] = (
                    acc_slc[pl.ds(r * 16, 16)] + tmp_slc[pl.ds(r * 16, 16)])
                return c2
            lax.fori_loop(0, DSL // 16, _r, 0)
            return carry
        lax.fori_loop(0, NS, _accp, 0)

        pltpu.sync_copy(acc_slc, deg_sh.at[pl.ds(rch * DCH + sid * DSL, DSL)])
        plsc.subcore_barrier()

    pltpu.sync_copy(deg_sh, dd)

    one16 = jnp.ones((16,), jnp.float32)

    def _p1(r, carry):
        dd[pl.ds(r * 16, 16)] = dd[pl.ds(r * 16, 16)] + one16
        return carry
    lax.fori_loop(0, NPAD // 16, _p1, 0)

    @pl.when(jnp.logical_and(cid == 0, sid == 0))
    def _write_deg():
        pltpu.sync_copy(dd, deg_out)

    def _dis(r, carry):
        dd[pl.ds(r * 16, 16)] = _fast_rsqrt(dd[pl.ds(r * 16, 16)])
        return carry
    lax.fori_loop(0, NPAD // 16, _dis, 0)

    tbase = jnp.where(cid == 0, sid * R0, NS * R0 + sid * R1)
    ngrp = jnp.where(cid == 0, R0 // 16, R1 // 16)
    bufs = (rows, rows_b)
    gsems = (gsem_a, gsem_b)
    ssems = (ssem_a, ssem_b)

    def _scale(buf, b):
        def _rowg(g16, c2):
            sv = sidx[b, pl.ds(g16 * 16, 16)]
            wv = ewb[b, pl.ds(g16 * 16, 16)]
            av = plsc.load_gather(dd, [sv]) * wv
            for u in range(16):
                spl = lax.broadcast(av[u], (16,))
                r = g16 * 16 + u
                for c8 in range(8):
                    buf[r, pl.ds(c8 * 16, 16)] = (
                        buf[r, pl.ds(c8 * 16, 16)] * spl)
            return c2
        lax.fori_loop(0, 8, _rowg, 0)

    NB = 16

    def _grp(g, carry):
        gb = tbase + g * NB
        i1 = pltpu.async_copy(src_hbm.at[pl.ds(gb, NB)], sidx, isem)
        i2 = pltpu.async_copy(dst_hbm.at[pl.ds(gb, NB)], didx, gsems[1])
        i3 = pltpu.async_copy(ew_hbm.at[pl.ds(gb, NB)], ewb, ssems[1])
        i1.wait()
        i2.wait()
        i3.wait()

        gather_d = pltpu.async_copy(x_hbm.at[sidx.at[0]], bufs[0], gsems[0])
        scat_d = [None, None]
        for b in range(NB):
            i, o = b % 2, (b + 1) % 2
            if b < NB - 1:
                if scat_d[o] is not None:
                    scat_d[o].wait()
                next_gather = pltpu.async_copy(
                    x_hbm.at[sidx.at[b + 1]], bufs[o], gsems[o])
            gather_d.wait()
            _scale(bufs[i], b)
            scat_d[i] = pltpu.async_copy(
                bufs[i], S_sh.at[didx.at[b]], ssems[i], add=True)
            if b < NB - 1:
                gather_d = next_gather
        scat_d[0].wait()
        scat_d[1].wait()
        return carry
    lax.fori_loop(0, ngrp, _grp, 0)

    plsc.subcore_barrier()

    exp = []
    for k in range(5):
        nrows = 128 if k < 4 else SLICE - 4 * 128
        off = sid * SLICE + k * 128
        exp.append(pltpu.async_copy(S_sh.at[pl.ds(off, nrows)],
                                    s_out.at[pl.ds(cid * N + off, nrows)],
                                    gsem_a))

    @pl.when(sid == NS - 1)
    def _export_tail():
        pltpu.sync_copy(S_sh.at[pl.ds(NS * SLICE, 16)],
                        s_out.at[pl.ds(cid * N + NS * SLICE, 16)])

    for d in exp:
        d.wait()


BLK = 1000


def _tc_body(s0_ref, s1_ref, x_ref, deg_ref, th_ref, o_ref):
    dis = lax.rsqrt(deg_ref[...])
    x = x_ref[...]
    w = dis * (s0_ref[...] + s1_ref[...] + dis * x)
    o_ref[...] = jnp.dot(w, th_ref[...], preferred_element_type=jnp.float32) * x


_tc_finish = pl.pallas_call(
    _tc_body,
    grid=(N // BLK,),
    in_specs=[
        pl.BlockSpec((BLK, D), lambda i: (i, 0)),
        pl.BlockSpec((BLK, D), lambda i: (i, 0)),
        pl.BlockSpec((BLK, D), lambda i: (i, 0)),
        pl.BlockSpec((BLK, 1), lambda i: (i, 0)),
        pl.BlockSpec((D, D), lambda i: (0, 0)),
    ],
    out_specs=pl.BlockSpec((BLK, D), lambda i: (i, 0)),
    out_shape=jax.ShapeDtypeStruct((N, D), jnp.float32),
)


def kernel(X, edge_index, edge_weight, theta):
    src = edge_index[0].astype(jnp.int32)
    dst = edge_index[1].astype(jnp.int32)
    ew = edge_weight.astype(jnp.float32)
    pad = EP - E
    src2d = jnp.concatenate([src, jnp.zeros((pad,), jnp.int32)]).reshape(ROWS_ALL, 128)
    dst2d = jnp.concatenate([dst, jnp.zeros((pad,), jnp.int32)]).reshape(ROWS_ALL, 128)
    ew2d = jnp.concatenate([ew, jnp.zeros((pad,), jnp.float32)]).reshape(ROWS_ALL, 128)

    dstf2d = jax.lax.bitcast_convert_type(dst2d, jnp.float32)
    s_cat, deg_pad = _sc_spmm(X, src2d, dst2d, dstf2d, ew2d)
    deg = deg_pad[:N].reshape(N, 1)
    return _tc_finish(s_cat[:N], s_cat[N:], X, deg, theta)
